# Initial kernel scaffold; baseline (speedup 1.0000x reference)
import jax
import jax.numpy as jnp
from jax.experimental import pallas as pl


def kernel(x, edge_index, W0, a_src0, a_dst0, b0, W1, a_src1, a_dst1, b1):
    raise NotImplementedError("write your pallas kernel here")


# speedup vs baseline: 26.0250x; 26.0250x over previous
import functools

import jax
import jax.numpy as jnp
from jax import lax
from jax.experimental import pallas as pl
from jax.experimental.pallas import tpu as pltpu
from jax.experimental.pallas import tpu_sc as plsc

N = 10000
D = 128
E = 320000

NC = 2
NS = 16
NW = NC * NS
CH = 128
CPT = 79
EPT = CH * CPT
EPAD = EPT * NW

RPT = N // NS
NP = 10240


def _proj_body(x_ref, w_ref, avs_ref, avd_ref, h_ref, as_ref, ad_ref, m_ref):
    h = jnp.dot(x_ref[...], w_ref[...], preferred_element_type=jnp.float32)
    h_ref[...] = h
    asv = jnp.sum(h * avs_ref[...], axis=1)
    adv = jnp.sum(h * avd_ref[...], axis=1)
    as_ref[...] = asv
    ad_ref[...] = adv
    b = jnp.max(asv) + jnp.max(adv)
    m = jnp.maximum(b, 0.2 * b)
    m_ref[...] = jnp.full((16,), m, jnp.float32)


def _proj(x, w, avs, avd):
    return pl.pallas_call(
        _proj_body,
        out_shape=[
            jax.ShapeDtypeStruct((N, D), jnp.float32),
            jax.ShapeDtypeStruct((N,), jnp.float32),
            jax.ShapeDtypeStruct((N,), jnp.float32),
            jax.ShapeDtypeStruct((16,), jnp.float32),
        ],
    )(x, w, avs, avd)


def _mid_body(acc_ref, den_ref, b0_ref, w1_ref, avs_ref, avd_ref,
              h2_ref, as_ref, ad_ref, m_ref):
    den = den_ref[0, :N] + den_ref[1, :N] + 1e-16
    out0 = (acc_ref[0] + acc_ref[1]) / den[:, None] + b0_ref[...]
    h1 = jnp.where(out0 > 0.0, out0,
                   jnp.exp(jnp.minimum(out0, 0.0)) - 1.0)
    h2 = jnp.dot(h1, w1_ref[...], preferred_element_type=jnp.float32)
    h2_ref[...] = h2
    asv = jnp.sum(h2 * avs_ref[...], axis=1)
    adv = jnp.sum(h2 * avd_ref[...], axis=1)
    as_ref[...] = asv
    ad_ref[...] = adv
    b = jnp.max(asv) + jnp.max(adv)
    m = jnp.maximum(b, 0.2 * b)
    m_ref[...] = jnp.full((16,), m, jnp.float32)


def _mid(acc, den, b0, w1, avs, avd):
    return pl.pallas_call(
        _mid_body,
        out_shape=[
            jax.ShapeDtypeStruct((N, D), jnp.float32),
            jax.ShapeDtypeStruct((N,), jnp.float32),
            jax.ShapeDtypeStruct((N,), jnp.float32),
            jax.ShapeDtypeStruct((16,), jnp.float32),
        ],
    )(acc, den, b0, w1, avs, avd)


def _fin_body(acc_ref, den_ref, b1_ref, out_ref):
    den = den_ref[0, :N] + den_ref[1, :N] + 1e-16
    out_ref[...] = (acc_ref[0] + acc_ref[1]) / den[:, None] + b1_ref[...]


def _fin(acc, den, b1):
    return pl.pallas_call(
        _fin_body,
        out_shape=jax.ShapeDtypeStruct((N, D), jnp.float32),
    )(acc, den, b1)


_MESH = plsc.VectorSubcoreMesh(core_axis_name="c", subcore_axis_name="s")


@functools.partial(
    pl.kernel,
    out_type=[
        jax.ShapeDtypeStruct((NC, N, D), jnp.float32),
        jax.ShapeDtypeStruct((NC, NP), jnp.float32),
    ],
    mesh=_MESH,
    scratch_types=[
        pltpu.VMEM((1, CH), jnp.int32),
        pltpu.VMEM((1, CH), jnp.int32),
        pltpu.VMEM((N,), jnp.float32),
        pltpu.VMEM((N,), jnp.float32),
        pltpu.VMEM((CH,), jnp.float32),
        pltpu.VMEM((CH, D), jnp.float32),
        pltpu.VMEM((1024,), jnp.float32),
        pltpu.VMEM((16,), jnp.float32),
        pltpu.VMEM_SHARED((N, D), jnp.float32),
        pltpu.VMEM_SHARED((NP,), jnp.float32),
        pltpu.SemaphoreType.DMA,
    ],
    compiler_params=pltpu.CompilerParams(needs_layout_passes=False),
)
def _edge(h_hbm, asl_hbm, adl_hbm, src_hbm, dst_hbm, m_hbm,
          acc_out, den_out,
          src_c, dst_c, asl, adl, exb, rows, zb, mv,
          acc_sh, den_sh, sem):
    c = lax.axis_index("c")
    s = lax.axis_index("s")
    wid = c * NS + s

    zero16 = jnp.zeros((16,), jnp.float32)

    def _zrow(r, carry):
        for kk in range(D // 16):
            rows[r, pl.ds(kk * 16, 16)] = zero16
        return carry
    lax.fori_loop(0, CH, _zrow, 0)

    def _zzb(i, carry):
        zb[pl.ds(i * 16, 16)] = zero16
        return carry
    lax.fori_loop(0, 1024 // 16, _zzb, 0)

    for q in range(5):
        pltpu.sync_copy(rows.at[pl.ds(0, 125)],
                        acc_sh.at[pl.ds(s * RPT + q * 125, 125)])

    @pl.when(s < 10)
    def _():
        pltpu.sync_copy(zb, den_sh.at[pl.ds(s * 1024, 1024)])

    pltpu.sync_copy(asl_hbm, asl)
    pltpu.sync_copy(adl_hbm, adl)
    pltpu.sync_copy(m_hbm, mv)

    plsc.subcore_barrier()

    base = wid * EPT
    m = mv[...]
    lanes = lax.broadcasted_iota(jnp.int32, (16,), 0)

    def _chunk(j, carry):
        off = base + j * CH
        pltpu.sync_copy(src_hbm.at[pl.ds(off, CH)], src_c.at[0])
        pltpu.sync_copy(dst_hbm.at[pl.ds(off, CH)], dst_c.at[0])

        cp = pltpu.async_copy(h_hbm.at[src_c.at[0]], rows, sem)

        for i in range(CH // 16):
            sv = src_c[0, pl.ds(i * 16, 16)]
            dv = dst_c[0, pl.ds(i * 16, 16)]
            a1 = plsc.load_gather(asl, [sv])
            a2 = plsc.load_gather(adl, [dv])
            e = a1 + a2
            e = jnp.maximum(e, 0.2 * e) - m
            ex = jnp.exp(e)
            pos = off + i * 16 + lanes
            ex = jnp.where(pos < E, ex, 0.0)
            exb[pl.ds(i * 16, 16)] = ex

        pltpu.sync_copy(exb, den_sh.at[dst_c.at[0]], add=True)

        cp.wait()

        def _scale(r, carry2):
            w = plsc.load_gather(exb, [jnp.full((16,), r, jnp.int32)])
            for kk in range(D // 16):
                rows[r, pl.ds(kk * 16, 16)] = rows[r, pl.ds(kk * 16, 16)] * w
            return carry2
        lax.fori_loop(0, CH, _scale, 0)

        pltpu.sync_copy(rows, acc_sh.at[dst_c.at[0]], add=True)
        return carry

    lax.fori_loop(0, CPT, _chunk, 0)

    plsc.subcore_barrier()

    pltpu.sync_copy(acc_sh.at[pl.ds(s * 624, 624)],
                    acc_out.at[c, pl.ds(s * 624, 624)])

    @pl.when(s == NS - 1)
    def _():
        pltpu.sync_copy(acc_sh.at[pl.ds(9984, 16)],
                        acc_out.at[c, pl.ds(9984, 16)])

    pltpu.sync_copy(den_sh.at[pl.ds(s * 640, 640)],
                    den_out.at[c, pl.ds(s * 640, 640)])


def kernel(x, edge_index, W0, a_src0, a_dst0, b0, W1, a_src1, a_dst1, b1):
    src = edge_index[0].astype(jnp.int32)
    dst = edge_index[1].astype(jnp.int32)
    npad = EPAD - E
    pad = (jnp.arange(npad, dtype=jnp.int32) * 37) % N
    srcp = jnp.concatenate([src, pad])
    dstp = jnp.concatenate([dst, pad])

    h, asv, adv, m0 = _proj(x, W0, a_src0, a_dst0)
    acc0, den0 = _edge(h, asv, adv, srcp, dstp, m0)
    h2, as1, ad1, m1 = _mid(acc0, den0, b0, W1, a_src1, a_dst1)
    acc1, den1 = _edge(h2, as1, ad1, srcp, dstp, m1)
    return _fin(acc1, den1, b1)


# speedup vs baseline: 34.8533x; 1.3392x over previous
import functools

import jax
import jax.numpy as jnp
from jax import lax
from jax.experimental import pallas as pl
from jax.experimental.pallas import tpu as pltpu
from jax.experimental.pallas import tpu_sc as plsc

N = 10000
D = 128
E = 320000

NC = 2
NS = 16
NW = NC * NS
CH = 128
CPT = 80
EPT = CH * CPT
EPAD = EPT * NW

RPT = N // NS
NP = 10240


def _proj_body(x_ref, w_ref, avs_ref, avd_ref, h_ref, as_ref, ad_ref, m_ref):
    h = jnp.dot(x_ref[...], w_ref[...], preferred_element_type=jnp.float32)
    h_ref[...] = h
    asv = jnp.sum(h * avs_ref[...], axis=1)
    adv = jnp.sum(h * avd_ref[...], axis=1)
    as_ref[...] = asv
    ad_ref[...] = adv
    b = jnp.max(asv) + jnp.max(adv)
    m = jnp.maximum(b, 0.2 * b)
    m_ref[...] = jnp.full((16,), m, jnp.float32)


def _proj(x, w, avs, avd):
    return pl.pallas_call(
        _proj_body,
        out_shape=[
            jax.ShapeDtypeStruct((N, D), jnp.float32),
            jax.ShapeDtypeStruct((N,), jnp.float32),
            jax.ShapeDtypeStruct((N,), jnp.float32),
            jax.ShapeDtypeStruct((16,), jnp.float32),
        ],
    )(x, w, avs, avd)


def _mid_body(acc_ref, den_ref, b0_ref, w1_ref, avs_ref, avd_ref,
              h2_ref, as_ref, ad_ref, m_ref):
    den = den_ref[0, :N] + den_ref[1, :N] + 1e-16
    out0 = (acc_ref[0] + acc_ref[1]) / den[:, None] + b0_ref[...]
    h1 = jnp.where(out0 > 0.0, out0,
                   jnp.exp(jnp.minimum(out0, 0.0)) - 1.0)
    h2 = jnp.dot(h1, w1_ref[...], preferred_element_type=jnp.float32)
    h2_ref[...] = h2
    asv = jnp.sum(h2 * avs_ref[...], axis=1)
    adv = jnp.sum(h2 * avd_ref[...], axis=1)
    as_ref[...] = asv
    ad_ref[...] = adv
    b = jnp.max(asv) + jnp.max(adv)
    m = jnp.maximum(b, 0.2 * b)
    m_ref[...] = jnp.full((16,), m, jnp.float32)


def _mid(acc, den, b0, w1, avs, avd):
    return pl.pallas_call(
        _mid_body,
        out_shape=[
            jax.ShapeDtypeStruct((N, D), jnp.float32),
            jax.ShapeDtypeStruct((N,), jnp.float32),
            jax.ShapeDtypeStruct((N,), jnp.float32),
            jax.ShapeDtypeStruct((16,), jnp.float32),
        ],
    )(acc, den, b0, w1, avs, avd)


def _fin_body(acc_ref, den_ref, b1_ref, out_ref):
    den = den_ref[0, :N] + den_ref[1, :N] + 1e-16
    out_ref[...] = (acc_ref[0] + acc_ref[1]) / den[:, None] + b1_ref[...]


def _fin(acc, den, b1):
    return pl.pallas_call(
        _fin_body,
        out_shape=jax.ShapeDtypeStruct((N, D), jnp.float32),
    )(acc, den, b1)


_MESH = plsc.VectorSubcoreMesh(core_axis_name="c", subcore_axis_name="s")


@functools.partial(
    pl.kernel,
    out_type=[
        jax.ShapeDtypeStruct((NC, N, D), jnp.float32),
        jax.ShapeDtypeStruct((NC, NP), jnp.float32),
    ],
    mesh=_MESH,
    scratch_types=[
        pltpu.VMEM((2, CH), jnp.int32),
        pltpu.VMEM((2, CH), jnp.int32),
        pltpu.VMEM((CH,), jnp.float32),
        pltpu.VMEM((CH,), jnp.float32),
        pltpu.VMEM((CH,), jnp.float32),
        pltpu.VMEM((CH, D), jnp.float32),
        pltpu.VMEM((CH, D), jnp.float32),
        pltpu.VMEM((16,), jnp.float32),
        pltpu.VMEM_SHARED((N, D), jnp.float32),
        pltpu.VMEM_SHARED((NP,), jnp.float32),
        pltpu.VMEM_SHARED((N,), jnp.float32),
        pltpu.VMEM_SHARED((N,), jnp.float32),
        pltpu.SemaphoreType.DMA,
        pltpu.SemaphoreType.DMA,
        pltpu.SemaphoreType.DMA,
        pltpu.SemaphoreType.DMA,
        pltpu.SemaphoreType.DMA,
    ],
    compiler_params=pltpu.CompilerParams(needs_layout_passes=False),
)
def _edge(h_hbm, asl_hbm, adl_hbm, src_hbm, dst_hbm, m_hbm,
          acc_out, den_out,
          srcb, dstb, a_s, a_d, exb, rows0, rows1, mv,
          acc_sh, den_sh, asv_sp, adv_sp, sg0, sg1, ss0, ss1, sa):
    c = lax.axis_index("c")
    s = lax.axis_index("s")
    wid = c * NS + s
    base = wid * EPT

    ROWS = (rows0, rows1)
    SG = (sg0, sg1)
    SS = (ss0, ss1)

    zero16 = jnp.zeros((16,), jnp.float32)

    def _zrow(r, carry):
        for kk in range(D // 16):
            rows0[r, pl.ds(kk * 16, 16)] = zero16
            rows1[r, pl.ds(kk * 16, 16)] = zero16
        return carry
    lax.fori_loop(0, CH, _zrow, 0)

    for i in range(CH // 16):
        exb[pl.ds(i * 16, 16)] = zero16

    for q in range(5):
        pltpu.sync_copy(rows0.at[pl.ds(0, 125)],
                        acc_sh.at[pl.ds(s * RPT + q * 125, 125)])

    @pl.when(s < 10)
    def _():
        for q in range(8):
            pltpu.sync_copy(exb, den_sh.at[pl.ds(s * 1024 + q * CH, CH)])

    @pl.when(s == 0)
    def _():
        pltpu.sync_copy(asl_hbm, asv_sp)
        pltpu.sync_copy(adl_hbm, adv_sp)
    pltpu.sync_copy(m_hbm, mv)

    plsc.subcore_barrier()

    m = mv[...]
    lanes = lax.broadcasted_iota(jnp.int32, (16,), 0)

    def load_idx(j, b):
        off = base + j * CH
        pltpu.sync_copy(src_hbm.at[pl.ds(off, CH)], srcb.at[b])
        pltpu.sync_copy(dst_hbm.at[pl.ds(off, CH)], dstb.at[b])

    def start_gather(b):
        pltpu.async_copy(h_hbm.at[srcb.at[b]], ROWS[b], SG[b])

    def wait_gather(b):
        pltpu.make_async_copy(h_hbm.at[srcb.at[b]], ROWS[b], SG[b]).wait()

    def start_scatter(b):
        pltpu.async_copy(ROWS[b], acc_sh.at[dstb.at[b]], SS[b], add=True)

    def wait_scatter(b):
        pltpu.make_async_copy(ROWS[b], acc_sh.at[dstb.at[b]], SS[b]).wait()

    def compute(j, b):
        off = base + j * CH
        cp1 = pltpu.async_copy(asv_sp.at[srcb.at[b]], a_s, sa)
        cp2 = pltpu.async_copy(adv_sp.at[dstb.at[b]], a_d, sa)
        cp1.wait()
        cp2.wait()
        for i in range(CH // 16):
            a1 = a_s[pl.ds(i * 16, 16)]
            a2 = a_d[pl.ds(i * 16, 16)]
            e = a1 + a2
            e = jnp.maximum(e, 0.2 * e) - m
            ex = jnp.exp(e)
            pos = off + i * 16 + lanes
            ex = jnp.where(pos < E, ex, 0.0)
            exb[pl.ds(i * 16, 16)] = ex
        pltpu.sync_copy(exb, den_sh.at[dstb.at[b]], add=True)
        rws = ROWS[b]

        def _scale(r, carry2):
            w = plsc.load_gather(exb, [jnp.full((16,), r, jnp.int32)])
            for kk in range(D // 16):
                rws[r, pl.ds(kk * 16, 16)] = rws[r, pl.ds(kk * 16, 16)] * w
            return carry2
        lax.fori_loop(0, CH, _scale, 0)

    load_idx(0, 0)
    start_gather(0)
    load_idx(1, 1)
    start_scatter(1)

    NT = CPT // 2

    def body(t, carry):
        j0 = 2 * t
        j1 = j0 + 1
        wait_scatter(1)
        load_idx(j1, 1)
        start_gather(1)
        wait_gather(0)
        compute(j0, 0)
        start_scatter(0)
        wait_gather(1)
        compute(j1, 1)
        wait_scatter(0)

        @pl.when(t + 1 < NT)
        def _():
            load_idx(j0 + 2, 0)
            start_gather(0)
        start_scatter(1)
        return carry

    lax.fori_loop(0, NT, body, 0)
    wait_scatter(1)

    plsc.subcore_barrier()

    pltpu.sync_copy(acc_sh.at[pl.ds(s * 624, 624)],
                    acc_out.at[c, pl.ds(s * 624, 624)])

    @pl.when(s == NS - 1)
    def _():
        pltpu.sync_copy(acc_sh.at[pl.ds(9984, 16)],
                        acc_out.at[c, pl.ds(9984, 16)])

    pltpu.sync_copy(den_sh.at[pl.ds(s * 640, 640)],
                    den_out.at[c, pl.ds(s * 640, 640)])


def kernel(x, edge_index, W0, a_src0, a_dst0, b0, W1, a_src1, a_dst1, b1):
    src = edge_index[0].astype(jnp.int32)
    dst = edge_index[1].astype(jnp.int32)
    npad = EPAD - E
    pad = (jnp.arange(npad, dtype=jnp.int32) * 37) % N
    srcp = jnp.concatenate([src, pad])
    dstp = jnp.concatenate([dst, pad])

    h, asv, adv, m0 = _proj(x, W0, a_src0, a_dst0)
    acc0, den0 = _edge(h, asv, adv, srcp, dstp, m0)
    h2, as1, ad1, m1 = _mid(acc0, den0, b0, W1, a_src1, a_dst1)
    acc1, den1 = _edge(h2, as1, ad1, srcp, dstp, m1)
    return _fin(acc1, den1, b1)


# speedup vs baseline: 46.1090x; 1.3229x over previous
import functools

import jax
import jax.numpy as jnp
from jax import lax
from jax.experimental import pallas as pl
from jax.experimental.pallas import tpu as pltpu
from jax.experimental.pallas import tpu_sc as plsc

N = 10000
D = 128
E = 320000

NC = 2
NS = 16
NW = NC * NS
CH = 128
CPT = 80
EPT = CH * CPT
EPAD = EPT * NW

RPT = N // NS
NP = 10240


def _proj_body(x_ref, w_ref, avs_ref, avd_ref, h_ref, as_ref, ad_ref, m_ref):
    h = jnp.dot(x_ref[...], w_ref[...], preferred_element_type=jnp.float32)
    h_ref[...] = h
    asv = jnp.sum(h * avs_ref[...], axis=1)
    adv = jnp.sum(h * avd_ref[...], axis=1)
    as_ref[...] = asv
    ad_ref[...] = adv
    b = jnp.max(asv) + jnp.max(adv)
    m = jnp.maximum(b, 0.2 * b)
    m_ref[...] = jnp.full((16,), m, jnp.float32)


def _proj(x, w, avs, avd):
    return pl.pallas_call(
        _proj_body,
        out_shape=[
            jax.ShapeDtypeStruct((N, D), jnp.float32),
            jax.ShapeDtypeStruct((N,), jnp.float32),
            jax.ShapeDtypeStruct((N,), jnp.float32),
            jax.ShapeDtypeStruct((16,), jnp.float32),
        ],
    )(x, w, avs, avd)


def _mid_body(acc_ref, den_ref, b0_ref, w1_ref, avs_ref, avd_ref,
              h2_ref, as_ref, ad_ref, m_ref):
    den = den_ref[0, :N] + den_ref[1, :N] + 1e-16
    out0 = (acc_ref[0] + acc_ref[1]) / den[:, None] + b0_ref[...]
    h1 = jnp.where(out0 > 0.0, out0,
                   jnp.exp(jnp.minimum(out0, 0.0)) - 1.0)
    h2 = jnp.dot(h1, w1_ref[...], preferred_element_type=jnp.float32)
    h2_ref[...] = h2
    asv = jnp.sum(h2 * avs_ref[...], axis=1)
    adv = jnp.sum(h2 * avd_ref[...], axis=1)
    as_ref[...] = asv
    ad_ref[...] = adv
    b = jnp.max(asv) + jnp.max(adv)
    m = jnp.maximum(b, 0.2 * b)
    m_ref[...] = jnp.full((16,), m, jnp.float32)


def _mid(acc, den, b0, w1, avs, avd):
    return pl.pallas_call(
        _mid_body,
        out_shape=[
            jax.ShapeDtypeStruct((N, D), jnp.float32),
            jax.ShapeDtypeStruct((N,), jnp.float32),
            jax.ShapeDtypeStruct((N,), jnp.float32),
            jax.ShapeDtypeStruct((16,), jnp.float32),
        ],
    )(acc, den, b0, w1, avs, avd)


def _fin_body(acc_ref, den_ref, b1_ref, out_ref):
    den = den_ref[0, :N] + den_ref[1, :N] + 1e-16
    out_ref[...] = (acc_ref[0] + acc_ref[1]) / den[:, None] + b1_ref[...]


def _fin(acc, den, b1):
    return pl.pallas_call(
        _fin_body,
        out_shape=jax.ShapeDtypeStruct((N, D), jnp.float32),
    )(acc, den, b1)


_MESH = plsc.VectorSubcoreMesh(core_axis_name="c", subcore_axis_name="s")

G = 8
NB = CPT // G


@functools.partial(
    pl.kernel,
    out_type=[
        jax.ShapeDtypeStruct((NC, N, D), jnp.float32),
        jax.ShapeDtypeStruct((NC, NP), jnp.float32),
    ],
    mesh=_MESH,
    scratch_types=[
        pltpu.VMEM((2, G, CH), jnp.int32),
        pltpu.VMEM((2, G, CH), jnp.int32),
        pltpu.VMEM((G, CH), jnp.float32),
        pltpu.VMEM((G, CH), jnp.float32),
        pltpu.VMEM((G, CH), jnp.float32),
        pltpu.VMEM((CH, D), jnp.float32),
        pltpu.VMEM((CH, D), jnp.float32),
        pltpu.VMEM((16,), jnp.float32),
        pltpu.VMEM_SHARED((N, D), jnp.float32),
        pltpu.VMEM_SHARED((NP,), jnp.float32),
        pltpu.VMEM_SHARED((N,), jnp.float32),
        pltpu.VMEM_SHARED((N,), jnp.float32),
        pltpu.SemaphoreType.DMA,
        pltpu.SemaphoreType.DMA,
        pltpu.SemaphoreType.DMA,
        pltpu.SemaphoreType.DMA,
        pltpu.SemaphoreType.DMA,
        pltpu.SemaphoreType.DMA,
        pltpu.SemaphoreType.DMA,
    ],
    compiler_params=pltpu.CompilerParams(needs_layout_passes=False),
)
def _edge(h_hbm, asl_hbm, adl_hbm, src_hbm, dst_hbm, m_hbm,
          acc_out, den_out,
          srcb, dstb, a_s, a_d, exb, rows0, rows1, mv,
          acc_sh, den_sh, asv_sp, adv_sp,
          sg0, sg1, ss0, ss1, sa, sd, si):
    c = lax.axis_index("c")
    s = lax.axis_index("s")
    wid = c * NS + s
    base_row = wid * CPT

    ROWS = (rows0, rows1)
    SG = (sg0, sg1)
    SS = (ss0, ss1)

    zero16 = jnp.zeros((16,), jnp.float32)

    def _zrow(r, carry):
        for kk in range(D // 16):
            rows0[r, pl.ds(kk * 16, 16)] = zero16
            rows1[r, pl.ds(kk * 16, 16)] = zero16
        return carry
    lax.fori_loop(0, CH, _zrow, 0)

    for g in range(G):
        for i in range(CH // 16):
            exb[g, pl.ds(i * 16, 16)] = zero16

    for q in range(5):
        pltpu.sync_copy(rows0.at[pl.ds(0, 125)],
                        acc_sh.at[pl.ds(s * RPT + q * 125, 125)])

    @pl.when(s < 10)
    def _():
        for q in range(8):
            pltpu.sync_copy(exb.at[0], den_sh.at[pl.ds(s * 1024 + q * CH, CH)])

    @pl.when(s == 0)
    def _():
        pltpu.sync_copy(asl_hbm, asv_sp)
        pltpu.sync_copy(adl_hbm, adv_sp)
    pltpu.sync_copy(m_hbm, mv)

    plsc.subcore_barrier()

    m = mv[...]
    lanes = lax.broadcasted_iota(jnp.int32, (16,), 0)

    def start_gather(bufref, sem, idxref):
        pltpu.async_copy(h_hbm.at[idxref], bufref, sem)

    def wait_gather(bufref, sem, idxref):
        pltpu.make_async_copy(h_hbm.at[idxref], bufref, sem).wait()

    def start_scatter(bufref, sem, idxref):
        pltpu.async_copy(bufref, acc_sh.at[idxref], sem, add=True)

    def wait_scatter(bufref, sem, idxref):
        pltpu.make_async_copy(bufref, acc_sh.at[idxref], sem).wait()

    pltpu.sync_copy(src_hbm.at[pl.ds(base_row, G)], srcb.at[0])
    pltpu.sync_copy(dst_hbm.at[pl.ds(base_row, G)], dstb.at[0])
    start_gather(rows0, sg0, srcb.at[0, 0])
    start_scatter(rows1, ss1, dstb.at[0, 1])

    def block(blk, carry):
        p = blk % 2
        sb = srcb.at[p]
        db = dstb.at[p]
        nsb = srcb.at[1 - p]
        ndb = dstb.at[1 - p]

        @pl.when(blk + 1 < NB)
        def _():
            nrow = base_row + (blk + 1) * G
            pltpu.async_copy(src_hbm.at[pl.ds(nrow, G)], nsb, si)
            pltpu.async_copy(dst_hbm.at[pl.ds(nrow, G)], ndb, si)

        for g in range(G):
            pltpu.async_copy(asv_sp.at[sb.at[g]], a_s.at[g], sa)
            pltpu.async_copy(adv_sp.at[db.at[g]], a_d.at[g], sa)
        for g in range(G):
            pltpu.make_async_copy(asv_sp.at[sb.at[g]], a_s.at[g], sa).wait()
            pltpu.make_async_copy(adv_sp.at[db.at[g]], a_d.at[g], sa).wait()

        for g in range(G):
            b = g % 2
            rws = ROWS[b]
            orws = ROWS[1 - b]
            exg = exb.at[g]
            off = (base_row + blk * G + g) * CH

            for i in range(CH // 16):
                e = a_s[g, pl.ds(i * 16, 16)] + a_d[g, pl.ds(i * 16, 16)]
                e = jnp.maximum(e, 0.2 * e) - m
                ex = jnp.exp(e)
                pos = off + i * 16 + lanes
                ex = jnp.where(pos < E, ex, 0.0)
                exb[g, pl.ds(i * 16, 16)] = ex

            pltpu.async_copy(exg, den_sh.at[db.at[g]], sd, add=True)

            wait_gather(rws, SG[b], db.at[g])

            @plsc.parallel_loop(0, CH, unroll=2)
            def _scale(r):
                w = plsc.load_gather(exg, [jnp.full((16,), r, jnp.int32)])
                for kk in range(D // 16):
                    rws[r, pl.ds(kk * 16, 16)] = rws[r, pl.ds(kk * 16, 16)] * w

            wait_scatter(orws, SS[1 - b], db.at[g])
            if g < G - 1:
                start_gather(orws, SG[1 - b], sb.at[g + 1])
            else:
                @pl.when(blk + 1 < NB)
                def _():
                    pltpu.make_async_copy(src_hbm.at[pl.ds(0, G)], nsb,
                                          si).wait()
                    pltpu.make_async_copy(dst_hbm.at[pl.ds(0, G)], ndb,
                                          si).wait()
                    start_gather(orws, SG[1 - b], nsb.at[0])

            start_scatter(rws, SS[b], db.at[g])

        for g in range(G):
            pltpu.make_async_copy(exb.at[g], den_sh.at[db.at[g]], sd).wait()
        return carry

    lax.fori_loop(0, NB, block, 0)

    pltpu.make_async_copy(rows1, acc_sh.at[dstb.at[1, G - 1]], ss1).wait()

    plsc.subcore_barrier()

    pltpu.sync_copy(acc_sh.at[pl.ds(s * 624, 624)],
                    acc_out.at[c, pl.ds(s * 624, 624)])

    @pl.when(s == NS - 1)
    def _():
        pltpu.sync_copy(acc_sh.at[pl.ds(9984, 16)],
                        acc_out.at[c, pl.ds(9984, 16)])

    pltpu.sync_copy(den_sh.at[pl.ds(s * 640, 640)],
                    den_out.at[c, pl.ds(s * 640, 640)])


def kernel(x, edge_index, W0, a_src0, a_dst0, b0, W1, a_src1, a_dst1, b1):
    src = edge_index[0].astype(jnp.int32)
    dst = edge_index[1].astype(jnp.int32)
    npad = EPAD - E
    pad = (jnp.arange(npad, dtype=jnp.int32) * 37) % N
    srcp = jnp.concatenate([src, pad]).reshape(EPAD // CH, CH)
    dstp = jnp.concatenate([dst, pad]).reshape(EPAD // CH, CH)

    h, asv, adv, m0 = _proj(x, W0, a_src0, a_dst0)
    acc0, den0 = _edge(h, asv, adv, srcp, dstp, m0)
    h2, as1, ad1, m1 = _mid(acc0, den0, b0, W1, a_src1, a_dst1)
    acc1, den1 = _edge(h2, as1, ad1, srcp, dstp, m1)
    return _fin(acc1, den1, b1)


# speedup vs baseline: 57.1109x; 1.2386x over previous
import functools

import jax
import jax.numpy as jnp
from jax import lax
from jax.experimental import pallas as pl
from jax.experimental.pallas import tpu as pltpu
from jax.experimental.pallas import tpu_sc as plsc

N = 10000
D = 128
E = 320000

NC = 2
NS = 16
NW = NC * NS
CH = 64
CPT = 160
EPT = CH * CPT
EPAD = EPT * NW

RPT = N // NS
NP = 10240


def _proj_body(x_ref, w_ref, avs_ref, avd_ref, h_ref, as_ref, ad_ref, m_ref):
    h = jnp.dot(x_ref[...], w_ref[...], preferred_element_type=jnp.float32)
    h_ref[...] = h
    asv = jnp.sum(h * avs_ref[...], axis=1)
    adv = jnp.sum(h * avd_ref[...], axis=1)
    as_ref[...] = asv
    ad_ref[...] = adv
    b = jnp.max(asv) + jnp.max(adv)
    m = jnp.maximum(b, 0.2 * b)
    m_ref[...] = jnp.full((16,), m, jnp.float32)


def _proj(x, w, avs, avd):
    return pl.pallas_call(
        _proj_body,
        out_shape=[
            jax.ShapeDtypeStruct((N, D), jnp.float32),
            jax.ShapeDtypeStruct((N,), jnp.float32),
            jax.ShapeDtypeStruct((N,), jnp.float32),
            jax.ShapeDtypeStruct((16,), jnp.float32),
        ],
    )(x, w, avs, avd)


def _mid_body(acc_ref, den_ref, b0_ref, w1_ref, avs_ref, avd_ref,
              h2_ref, as_ref, ad_ref, m_ref):
    den = den_ref[0, :N] + den_ref[1, :N] + 1e-16
    out0 = (acc_ref[0] + acc_ref[1]) / den[:, None] + b0_ref[...]
    h1 = jnp.where(out0 > 0.0, out0,
                   jnp.exp(jnp.minimum(out0, 0.0)) - 1.0)
    h2 = jnp.dot(h1, w1_ref[...], preferred_element_type=jnp.float32)
    h2_ref[...] = h2
    asv = jnp.sum(h2 * avs_ref[...], axis=1)
    adv = jnp.sum(h2 * avd_ref[...], axis=1)
    as_ref[...] = asv
    ad_ref[...] = adv
    b = jnp.max(asv) + jnp.max(adv)
    m = jnp.maximum(b, 0.2 * b)
    m_ref[...] = jnp.full((16,), m, jnp.float32)


def _mid(acc, den, b0, w1, avs, avd):
    return pl.pallas_call(
        _mid_body,
        out_shape=[
            jax.ShapeDtypeStruct((N, D), jnp.float32),
            jax.ShapeDtypeStruct((N,), jnp.float32),
            jax.ShapeDtypeStruct((N,), jnp.float32),
            jax.ShapeDtypeStruct((16,), jnp.float32),
        ],
    )(acc, den, b0, w1, avs, avd)


def _fin_body(acc_ref, den_ref, b1_ref, out_ref):
    den = den_ref[0, :N] + den_ref[1, :N] + 1e-16
    out_ref[...] = (acc_ref[0] + acc_ref[1]) / den[:, None] + b1_ref[...]


def _fin(acc, den, b1):
    return pl.pallas_call(
        _fin_body,
        out_shape=jax.ShapeDtypeStruct((N, D), jnp.float32),
    )(acc, den, b1)


_MESH = plsc.VectorSubcoreMesh(core_axis_name="c", subcore_axis_name="s")

G = 8
NB = CPT // G
NBUF = 4


@functools.partial(
    pl.kernel,
    out_type=[
        jax.ShapeDtypeStruct((NC, N, D), jnp.float32),
        jax.ShapeDtypeStruct((NC, NP), jnp.float32),
    ],
    mesh=_MESH,
    scratch_types=[
        pltpu.VMEM((2, G, CH), jnp.int32),
        pltpu.VMEM((2, G, CH), jnp.int32),
        pltpu.VMEM((G, CH), jnp.float32),
        pltpu.VMEM((G, CH), jnp.float32),
        pltpu.VMEM((G, CH), jnp.float32),
        pltpu.VMEM((CH, D), jnp.float32),
        pltpu.VMEM((CH, D), jnp.float32),
        pltpu.VMEM((CH, D), jnp.float32),
        pltpu.VMEM((CH, D), jnp.float32),
        pltpu.VMEM((16,), jnp.float32),
        pltpu.VMEM_SHARED((N, D), jnp.float32),
        pltpu.VMEM_SHARED((NP,), jnp.float32),
        pltpu.VMEM_SHARED((N,), jnp.float32),
        pltpu.VMEM_SHARED((N,), jnp.float32),
        pltpu.SemaphoreType.DMA,
        pltpu.SemaphoreType.DMA,
        pltpu.SemaphoreType.DMA,
        pltpu.SemaphoreType.DMA,
        pltpu.SemaphoreType.DMA,
        pltpu.SemaphoreType.DMA,
        pltpu.SemaphoreType.DMA,
        pltpu.SemaphoreType.DMA,
        pltpu.SemaphoreType.DMA,
        pltpu.SemaphoreType.DMA,
        pltpu.SemaphoreType.DMA,
    ],
    compiler_params=pltpu.CompilerParams(needs_layout_passes=False),
)
def _edge(h_hbm, asl_hbm, adl_hbm, src_hbm, dst_hbm, m_hbm,
          acc_out, den_out,
          srcb, dstb, a_s, a_d, exb, rows0, rows1, rows2, rows3, mv,
          acc_sh, den_sh, asv_sp, adv_sp,
          sg0, sg1, sg2, sg3, ss0, ss1, ss2, ss3, sa, sd, si):
    c = lax.axis_index("c")
    s = lax.axis_index("s")
    wid = c * NS + s
    base_row = wid * CPT

    ROWS = (rows0, rows1, rows2, rows3)
    SG = (sg0, sg1, sg2, sg3)
    SS = (ss0, ss1, ss2, ss3)

    zero16 = jnp.zeros((16,), jnp.float32)

    def _zrow(r, carry):
        for kk in range(D // 16):
            rows0[r, pl.ds(kk * 16, 16)] = zero16
            rows1[r, pl.ds(kk * 16, 16)] = zero16
            rows2[r, pl.ds(kk * 16, 16)] = zero16
            rows3[r, pl.ds(kk * 16, 16)] = zero16
        return carry
    lax.fori_loop(0, CH, _zrow, 0)

    for g in range(G):
        for i in range(CH // 16):
            exb[g, pl.ds(i * 16, 16)] = zero16

    for q in range(10):
        pltpu.sync_copy(rows0.at[pl.ds(0, 62)],
                        acc_sh.at[pl.ds(s * RPT + q * 62, 62)])
    pltpu.sync_copy(rows0.at[pl.ds(0, 5)],
                    acc_sh.at[pl.ds(s * RPT + 620, 5)])

    @pl.when(s < 10)
    def _():
        for q in range(16):
            pltpu.sync_copy(exb.at[0], den_sh.at[pl.ds(s * 1024 + q * CH, CH)])

    @pl.when(s == 0)
    def _():
        pltpu.sync_copy(asl_hbm, asv_sp)
        pltpu.sync_copy(adl_hbm, adv_sp)
    pltpu.sync_copy(m_hbm, mv)

    plsc.subcore_barrier()

    m = mv[...]
    lanes = lax.broadcasted_iota(jnp.int32, (16,), 0)

    def start_gather(bufref, sem, idxref):
        pltpu.async_copy(h_hbm.at[idxref], bufref, sem)

    def wait_gather(bufref, sem, idxref):
        pltpu.make_async_copy(h_hbm.at[idxref], bufref, sem).wait()

    def start_scatter(bufref, sem, idxref):
        pltpu.async_copy(bufref, acc_sh.at[idxref], sem, add=True)

    def wait_scatter(bufref, sem, idxref):
        pltpu.make_async_copy(bufref, acc_sh.at[idxref], sem).wait()

    pltpu.sync_copy(src_hbm.at[pl.ds(base_row, G)], srcb.at[0])
    pltpu.sync_copy(dst_hbm.at[pl.ds(base_row, G)], dstb.at[0])
    start_gather(rows0, sg0, srcb.at[0, 0])
    start_gather(rows1, sg1, srcb.at[0, 1])
    start_scatter(rows2, ss2, dstb.at[0, 2])
    start_scatter(rows3, ss3, dstb.at[0, 3])

    def block(blk, carry):
        p = blk % 2
        sb = srcb.at[p]
        db = dstb.at[p]
        nsb = srcb.at[1 - p]
        ndb = dstb.at[1 - p]

        @pl.when(blk + 1 < NB)
        def _():
            nrow = base_row + (blk + 1) * G
            pltpu.async_copy(src_hbm.at[pl.ds(nrow, G)], nsb, si)
            pltpu.async_copy(dst_hbm.at[pl.ds(nrow, G)], ndb, si)

        for g in range(G):
            pltpu.async_copy(asv_sp.at[sb.at[g]], a_s.at[g], sa)
            pltpu.async_copy(adv_sp.at[db.at[g]], a_d.at[g], sa)
        for g in range(G):
            pltpu.make_async_copy(asv_sp.at[sb.at[g]], a_s.at[g], sa).wait()
            pltpu.make_async_copy(adv_sp.at[db.at[g]], a_d.at[g], sa).wait()

        for g in range(G):
            b = g % NBUF
            rws = ROWS[b]
            exg = exb.at[g]
            off = (base_row + blk * G + g) * CH

            for i in range(CH // 16):
                e = a_s[g, pl.ds(i * 16, 16)] + a_d[g, pl.ds(i * 16, 16)]
                e = jnp.maximum(e, 0.2 * e) - m
                ex = jnp.exp(e)
                pos = off + i * 16 + lanes
                ex = jnp.where(pos < E, ex, 0.0)
                exb[g, pl.ds(i * 16, 16)] = ex

            pltpu.async_copy(exg, den_sh.at[db.at[g]], sd, add=True)

            wait_gather(rws, SG[b], sb.at[g])

            @plsc.parallel_loop(0, CH, unroll=2)
            def _scale(r):
                w = plsc.load_gather(exg, [jnp.full((16,), r, jnp.int32)])
                for kk in range(D // 16):
                    rws[r, pl.ds(kk * 16, 16)] = rws[r, pl.ds(kk * 16, 16)] * w

            fb = (g + 2) % NBUF
            fbuf = ROWS[fb]
            wait_scatter(fbuf, SS[fb], db.at[g])
            if g == G - 2:
                @pl.when(blk + 1 < NB)
                def _():
                    pltpu.make_async_copy(src_hbm.at[pl.ds(0, G)], nsb,
                                          si).wait()
                    pltpu.make_async_copy(dst_hbm.at[pl.ds(0, G)], ndb,
                                          si).wait()
            if g < G - 2:
                start_gather(fbuf, SG[fb], sb.at[g + 2])
            else:
                @pl.when(blk + 1 < NB)
                def _():
                    start_gather(fbuf, SG[fb], nsb.at[g + 2 - G])

            start_scatter(rws, SS[b], db.at[g])

        for g in range(G):
            pltpu.make_async_copy(exb.at[g], den_sh.at[db.at[g]], sd).wait()
        return carry

    lax.fori_loop(0, NB, block, 0)

    pltpu.make_async_copy(rows2, acc_sh.at[dstb.at[1, G - 2]], ss2).wait()
    pltpu.make_async_copy(rows3, acc_sh.at[dstb.at[1, G - 1]], ss3).wait()

    plsc.subcore_barrier()

    pltpu.sync_copy(acc_sh.at[pl.ds(s * 624, 624)],
                    acc_out.at[c, pl.ds(s * 624, 624)])

    @pl.when(s == NS - 1)
    def _():
        pltpu.sync_copy(acc_sh.at[pl.ds(9984, 16)],
                        acc_out.at[c, pl.ds(9984, 16)])

    pltpu.sync_copy(den_sh.at[pl.ds(s * 640, 640)],
                    den_out.at[c, pl.ds(s * 640, 640)])


def kernel(x, edge_index, W0, a_src0, a_dst0, b0, W1, a_src1, a_dst1, b1):
    src = edge_index[0].astype(jnp.int32)
    dst = edge_index[1].astype(jnp.int32)
    npad = EPAD - E
    pad = (jnp.arange(npad, dtype=jnp.int32) * 37) % N
    srcp = jnp.concatenate([src, pad]).reshape(EPAD // CH, CH)
    dstp = jnp.concatenate([dst, pad]).reshape(EPAD // CH, CH)

    h, asv, adv, m0 = _proj(x, W0, a_src0, a_dst0)
    acc0, den0 = _edge(h, asv, adv, srcp, dstp, m0)
    h2, as1, ad1, m1 = _mid(acc0, den0, b0, W1, a_src1, a_dst1)
    acc1, den1 = _edge(h2, as1, ad1, srcp, dstp, m1)
    return _fin(acc1, den1, b1)
